# baseline (speedup 1.0000x reference)
import functools

import jax
import jax.numpy as jnp
from jax import lax
from jax.experimental import pallas as pl
from jax.experimental.pallas import tpu as pltpu
from jax.experimental.pallas import tpu_sc as plsc

NN = 10000
FE = 128
HF = 64
EE = 320000
NP = 10240
EP = 323584
B = 128
NB = 158
RPT = NP // 16
CH = RPT // 2

_MESH = dict(core_axis_name="c", subcore_axis_name="s")


def _agg_body(src_hbm, dst_hbm, zpf_hbm, zeros_hbm, out_hbm, src_v, dst_v,
              row_v, buf_v, acc_sh, sem):
    c = lax.axis_index("c")
    t = lax.axis_index("s")
    pltpu.sync_copy(src_hbm.at[c, t], src_v)
    pltpu.sync_copy(dst_hbm.at[t], dst_v)
    pltpu.sync_copy(zeros_hbm, buf_v)
    for k in range(2):
        pltpu.sync_copy(buf_v, acc_sh.at[pl.ds(t * RPT + k * CH, CH)])
    plsc.subcore_barrier()

    pltpu.async_copy(zpf_hbm.at[src_v.at[0]], row_v.at[0], sem.at[0])
    pltpu.async_copy(zpf_hbm.at[src_v.at[1]], row_v.at[1], sem.at[1])

    def group(g, carry):
        for b in range(2):
            j = g * 2 + b
            pltpu.make_async_copy(
                zpf_hbm.at[src_v.at[j]], row_v.at[b], sem.at[b]).wait()
            pltpu.sync_copy(row_v.at[b], acc_sh.at[dst_v.at[j]], add=True)

            @pl.when(j + 2 < NB)
            def _():
                pltpu.async_copy(
                    zpf_hbm.at[src_v.at[j + 2]], row_v.at[b], sem.at[b])
        return carry

    lax.fori_loop(0, NB // 2, group, 0)
    plsc.subcore_barrier()
    for k in range(2):
        sl = pl.ds(t * RPT + k * CH, CH)
        pltpu.sync_copy(acc_sh.at[sl], buf_v)
        pltpu.sync_copy(buf_v, out_hbm.at[sl, c])


@functools.partial(
    pl.kernel,
    mesh=plsc.VectorSubcoreMesh(**_MESH),
    out_type=jax.ShapeDtypeStruct((NP, 2, HF), jnp.float32),
    scratch_types=[
        pltpu.VMEM((NB, B), jnp.int32),
        pltpu.VMEM((NB, B), jnp.int32),
        pltpu.VMEM((2, B, HF), jnp.float32),
        pltpu.VMEM((CH, HF), jnp.float32),
        pltpu.VMEM_SHARED((NP, HF), jnp.float32),
        pltpu.SemaphoreType.DMA((2,)),
    ],
    compiler_params=pltpu.CompilerParams(use_tc_tiling_on_sc=False),
)
def _agg_call(*args):
    _agg_body(*args)


NBD = EP // (32 * B)
HD = 16


def _deg_body(dst_hbm, ones_hbm, zeros_hbm, out_hbm, dst_v, ones_v, buf_v,
              acc_sh):
    c = lax.axis_index("c")
    t = lax.axis_index("s")
    wid = t * 2 + c
    pltpu.sync_copy(dst_hbm.at[wid], dst_v)
    pltpu.sync_copy(ones_hbm, ones_v)
    pltpu.sync_copy(zeros_hbm, buf_v)
    pltpu.sync_copy(buf_v, acc_sh.at[pl.ds(t * RPT, RPT)])
    plsc.subcore_barrier()

    def body(j, carry):
        pltpu.sync_copy(ones_v, acc_sh.at[dst_v.at[j]], add=True)
        return carry

    lax.fori_loop(0, NBD, body, 0)
    plsc.subcore_barrier()
    sl = pl.ds(t * RPT, RPT)
    pltpu.sync_copy(acc_sh.at[sl], buf_v)
    pltpu.sync_copy(buf_v, out_hbm.at[sl, c])


@functools.partial(
    pl.kernel,
    mesh=plsc.VectorSubcoreMesh(**_MESH),
    out_type=jax.ShapeDtypeStruct((NP, 2, HD), jnp.float32),
    scratch_types=[
        pltpu.VMEM((NBD, B), jnp.int32),
        pltpu.VMEM((B, HD), jnp.float32),
        pltpu.VMEM((RPT, HD), jnp.float32),
        pltpu.VMEM_SHARED((NP, HD), jnp.float32),
    ],
    compiler_params=pltpu.CompilerParams(use_tc_tiling_on_sc=False),
)
def _deg_call(*args):
    _deg_body(*args)


_BM = 1280


def _prep_body(xp_ref, w_ref, dinv_ref, zp_ref):
    z = jnp.dot(xp_ref[...], w_ref[...],
                preferred_element_type=jnp.float32,
                precision=lax.Precision.HIGHEST)
    zp_ref[...] = z * dinv_ref[...]


def _prep_call(xp, w, dinv):
    return pl.pallas_call(
        _prep_body,
        grid=(NP // _BM,),
        in_specs=[
            pl.BlockSpec((_BM, FE), lambda i: (i, 0)),
            pl.BlockSpec((FE, FE), lambda i: (0, 0)),
            pl.BlockSpec((_BM, 1), lambda i: (i, 0)),
        ],
        out_specs=pl.BlockSpec((_BM, FE), lambda i: (i, 0)),
        out_shape=jax.ShapeDtypeStruct((NP, FE), jnp.float32),
    )(xp, w, dinv)


def _next_body(agg_ref, zp_ref, dinv_ref, b_ref, w_ref, out_ref):
    dinv = dinv_ref[...]
    h = jnp.maximum((agg_ref[...] + zp_ref[...]) * dinv + b_ref[...], 0.0)
    z = jnp.dot(h, w_ref[...], preferred_element_type=jnp.float32,
                precision=lax.Precision.HIGHEST)
    out_ref[...] = z * dinv


def _next_call(agg, zp, dinv, b, w):
    return pl.pallas_call(
        _next_body,
        grid=(NP // _BM,),
        in_specs=[
            pl.BlockSpec((_BM, FE), lambda i: (i, 0)),
            pl.BlockSpec((_BM, FE), lambda i: (i, 0)),
            pl.BlockSpec((_BM, 1), lambda i: (i, 0)),
            pl.BlockSpec((1, FE), lambda i: (0, 0)),
            pl.BlockSpec((FE, FE), lambda i: (0, 0)),
        ],
        out_specs=pl.BlockSpec((_BM, FE), lambda i: (i, 0)),
        out_shape=jax.ShapeDtypeStruct((NP, FE), jnp.float32),
    )(agg, zp, dinv, b, w)


_BMF = 1000


def _fin_body(agg_ref, zp_ref, dinv_ref, b_ref, out_ref):
    h = (agg_ref[...] + zp_ref[...]) * dinv_ref[...] + b_ref[...]
    out_ref[...] = jnp.maximum(h, 0.0)


def _fin_call(agg, zp, dinv, b):
    return pl.pallas_call(
        _fin_body,
        grid=(NN // _BMF,),
        in_specs=[
            pl.BlockSpec((_BMF, FE), lambda i: (i, 0)),
            pl.BlockSpec((_BMF, FE), lambda i: (i, 0)),
            pl.BlockSpec((_BMF, 1), lambda i: (i, 0)),
            pl.BlockSpec((1, FE), lambda i: (0, 0)),
        ],
        out_specs=pl.BlockSpec((_BMF, FE), lambda i: (i, 0)),
        out_shape=jax.ShapeDtypeStruct((NN, FE), jnp.float32),
    )(agg, zp, dinv, b)


_BA = 400


def _rec_body(si_ref, sj_ref, out_ref):
    out_ref[...] = lax.dot_general(
        si_ref[...], sj_ref[...], (((1,), (1,)), ((), ())),
        precision=lax.Precision.HIGHEST,
        preferred_element_type=jnp.float32)


def _rec_call(s):
    return pl.pallas_call(
        _rec_body,
        grid=(NN // _BA,),
        in_specs=[
            pl.BlockSpec((_BA, FE), lambda i: (i, 0)),
            pl.BlockSpec((NN, FE), lambda i: (0, 0)),
        ],
        out_specs=pl.BlockSpec((_BA, NN), lambda i: (i, 0)),
        out_shape=jax.ShapeDtypeStruct((NN, NN), jnp.float32),
    )(s, s)


def kernel(x, edge_index, W1e, b1e, W2e, b2e, Wa1, ba1, Wa2, ba2, Ws1, bs1):
    src = edge_index[0].astype(jnp.int32)
    dst = edge_index[1].astype(jnp.int32)
    pad = EP - EE
    src_p = jnp.concatenate([src, jnp.zeros((pad,), jnp.int32)])
    dst_p = jnp.concatenate([dst, jnp.full((pad,), NN, jnp.int32)])
    srcs = jnp.stack([src_p * 2, src_p * 2 + 1]).reshape(2, 16, NB, B)
    dst3 = dst_p.reshape(16, NB, B)
    dstd = dst_p.reshape(32, NBD, B)
    xp = jnp.pad(x, ((0, NP - NN), (0, 0)))
    zeros = jnp.zeros((CH, HF), jnp.float32)
    ones16 = jnp.ones((B, HD), jnp.float32)
    zeros16 = jnp.zeros((RPT, HD), jnp.float32)

    def agg(zp):
        a = _agg_call(srcs, dst3, zp.reshape(2 * NP, HF), zeros)
        return a.reshape(NP, FE)

    degp = _deg_call(dstd, ones16, zeros16)
    deg = degp[:, 0, 0] + degp[:, 1, 0] + 1.0
    dinv = lax.rsqrt(jnp.maximum(deg, 1.0)).reshape(NP, 1)

    zp1 = _prep_call(xp, W1e, dinv)
    zp2 = _next_call(agg(zp1), zp1, dinv, b1e.reshape(1, FE), W2e)
    agg2 = agg(zp2)
    zp5 = _next_call(agg2, zp2, dinv, b2e.reshape(1, FE), Ws1)
    zp3 = _next_call(agg2, zp2, dinv, b2e.reshape(1, FE), Wa1)
    s = _fin_call(agg(zp5), zp5, dinv, bs1.reshape(1, FE))
    A_hat = _rec_call(s)
    zp4 = _next_call(agg(zp3), zp3, dinv, ba1.reshape(1, FE), Wa2)
    X_hat = _fin_call(agg(zp4), zp4, dinv, ba2.reshape(1, FE))
    return (A_hat, X_hat)


# speedup vs baseline: 1.0487x; 1.0487x over previous
import functools

import jax
import jax.numpy as jnp
from jax import lax
from jax.experimental import pallas as pl
from jax.experimental.pallas import tpu as pltpu
from jax.experimental.pallas import tpu_sc as plsc

NN = 10000
FE = 128
HF = 64
EE = 320000
NP = 10240
EP = 323584
B = 128
NB = 158
KR = 3
RPT = NP // 16
CH = RPT // 2

_MESH = dict(core_axis_name="c", subcore_axis_name="s")


def _agg_body(src_hbm, dst_hbm, zpf_hbm, zeros_hbm, out_hbm, src_v, dst_v,
              row_v, buf_v, acc_sh, sem):
    c = lax.axis_index("c")
    t = lax.axis_index("s")
    pltpu.sync_copy(src_hbm.at[c, t], src_v)
    pltpu.sync_copy(dst_hbm.at[t], dst_v)
    pltpu.sync_copy(zeros_hbm, buf_v)
    for k in range(2):
        pltpu.sync_copy(buf_v, acc_sh.at[pl.ds(t * RPT + k * CH, CH)])
    plsc.subcore_barrier()

    NBR = (NB // KR) * KR
    for b in range(KR):
        pltpu.async_copy(zpf_hbm.at[src_v.at[b]], row_v.at[b], sem.at[b])

    def group(g, carry):
        for b in range(KR):
            j = g * KR + b
            pltpu.make_async_copy(
                zpf_hbm.at[src_v.at[j]], row_v.at[b], sem.at[b]).wait()
            pltpu.sync_copy(row_v.at[b], acc_sh.at[dst_v.at[j]], add=True)

            @pl.when(j + KR < NBR)
            def _():
                pltpu.async_copy(
                    zpf_hbm.at[src_v.at[j + KR]], row_v.at[b], sem.at[b])
        return carry

    lax.fori_loop(0, NB // KR, group, 0)
    for j in range(NBR, NB):
        pltpu.sync_copy(zpf_hbm.at[src_v.at[j]], row_v.at[0])
        pltpu.sync_copy(row_v.at[0], acc_sh.at[dst_v.at[j]], add=True)
    plsc.subcore_barrier()
    for k in range(2):
        sl = pl.ds(t * RPT + k * CH, CH)
        pltpu.sync_copy(acc_sh.at[sl], buf_v)
        pltpu.sync_copy(buf_v, out_hbm.at[sl, c])


@functools.partial(
    pl.kernel,
    mesh=plsc.VectorSubcoreMesh(**_MESH),
    out_type=jax.ShapeDtypeStruct((NP, 2, HF), jnp.float32),
    scratch_types=[
        pltpu.VMEM((NB, B), jnp.int32),
        pltpu.VMEM((NB, B), jnp.int32),
        pltpu.VMEM((KR, B, HF), jnp.float32),
        pltpu.VMEM((CH, HF), jnp.float32),
        pltpu.VMEM_SHARED((NP, HF), jnp.float32),
        pltpu.SemaphoreType.DMA((KR,)),
    ],
    compiler_params=pltpu.CompilerParams(use_tc_tiling_on_sc=False),
)
def _agg_call(*args):
    _agg_body(*args)


NBD = EP // (32 * B)
HD = 16


def _deg_body(dst_hbm, ones_hbm, zeros_hbm, out_hbm, dst_v, ones_v, buf_v,
              acc_sh):
    c = lax.axis_index("c")
    t = lax.axis_index("s")
    wid = t * 2 + c
    pltpu.sync_copy(dst_hbm.at[wid], dst_v)
    pltpu.sync_copy(ones_hbm, ones_v)
    pltpu.sync_copy(zeros_hbm, buf_v)
    pltpu.sync_copy(buf_v, acc_sh.at[pl.ds(t * RPT, RPT)])
    plsc.subcore_barrier()

    def body(j, carry):
        pltpu.sync_copy(ones_v, acc_sh.at[dst_v.at[j]], add=True)
        return carry

    lax.fori_loop(0, NBD, body, 0)
    plsc.subcore_barrier()
    sl = pl.ds(t * RPT, RPT)
    pltpu.sync_copy(acc_sh.at[sl], buf_v)
    pltpu.sync_copy(buf_v, out_hbm.at[sl, c])


@functools.partial(
    pl.kernel,
    mesh=plsc.VectorSubcoreMesh(**_MESH),
    out_type=jax.ShapeDtypeStruct((NP, 2, HD), jnp.float32),
    scratch_types=[
        pltpu.VMEM((NBD, B), jnp.int32),
        pltpu.VMEM((B, HD), jnp.float32),
        pltpu.VMEM((RPT, HD), jnp.float32),
        pltpu.VMEM_SHARED((NP, HD), jnp.float32),
    ],
    compiler_params=pltpu.CompilerParams(use_tc_tiling_on_sc=False),
)
def _deg_call(*args):
    _deg_body(*args)


_BM = 1280


def _prep_body(xp_ref, w_ref, dinv_ref, zp_ref):
    z = jnp.dot(xp_ref[...], w_ref[...],
                preferred_element_type=jnp.float32,
                precision=lax.Precision.HIGHEST)
    zp_ref[...] = z * dinv_ref[...]


def _prep_call(xp, w, dinv):
    return pl.pallas_call(
        _prep_body,
        grid=(NP // _BM,),
        in_specs=[
            pl.BlockSpec((_BM, FE), lambda i: (i, 0)),
            pl.BlockSpec((FE, FE), lambda i: (0, 0)),
            pl.BlockSpec((_BM, 1), lambda i: (i, 0)),
        ],
        out_specs=pl.BlockSpec((_BM, FE), lambda i: (i, 0)),
        out_shape=jax.ShapeDtypeStruct((NP, FE), jnp.float32),
    )(xp, w, dinv)


def _next_body(agg_ref, zp_ref, dinv_ref, b_ref, w_ref, out_ref):
    dinv = dinv_ref[...]
    h = jnp.maximum((agg_ref[...] + zp_ref[...]) * dinv + b_ref[...], 0.0)
    z = jnp.dot(h, w_ref[...], preferred_element_type=jnp.float32,
                precision=lax.Precision.HIGHEST)
    out_ref[...] = z * dinv


def _next_call(agg, zp, dinv, b, w):
    return pl.pallas_call(
        _next_body,
        grid=(NP // _BM,),
        in_specs=[
            pl.BlockSpec((_BM, FE), lambda i: (i, 0)),
            pl.BlockSpec((_BM, FE), lambda i: (i, 0)),
            pl.BlockSpec((_BM, 1), lambda i: (i, 0)),
            pl.BlockSpec((1, FE), lambda i: (0, 0)),
            pl.BlockSpec((FE, FE), lambda i: (0, 0)),
        ],
        out_specs=pl.BlockSpec((_BM, FE), lambda i: (i, 0)),
        out_shape=jax.ShapeDtypeStruct((NP, FE), jnp.float32),
    )(agg, zp, dinv, b, w)


_BMF = 1000


def _fin_body(agg_ref, zp_ref, dinv_ref, b_ref, out_ref):
    h = (agg_ref[...] + zp_ref[...]) * dinv_ref[...] + b_ref[...]
    out_ref[...] = jnp.maximum(h, 0.0)


def _fin_call(agg, zp, dinv, b):
    return pl.pallas_call(
        _fin_body,
        grid=(NN // _BMF,),
        in_specs=[
            pl.BlockSpec((_BMF, FE), lambda i: (i, 0)),
            pl.BlockSpec((_BMF, FE), lambda i: (i, 0)),
            pl.BlockSpec((_BMF, 1), lambda i: (i, 0)),
            pl.BlockSpec((1, FE), lambda i: (0, 0)),
        ],
        out_specs=pl.BlockSpec((_BMF, FE), lambda i: (i, 0)),
        out_shape=jax.ShapeDtypeStruct((NN, FE), jnp.float32),
    )(agg, zp, dinv, b)


_BA = 400


def _rec_body(si_ref, sj_ref, out_ref):
    out_ref[...] = lax.dot_general(
        si_ref[...], sj_ref[...], (((1,), (1,)), ((), ())),
        precision=lax.Precision.HIGHEST,
        preferred_element_type=jnp.float32)


def _rec_call(s):
    return pl.pallas_call(
        _rec_body,
        grid=(NN // _BA,),
        in_specs=[
            pl.BlockSpec((_BA, FE), lambda i: (i, 0)),
            pl.BlockSpec((NN, FE), lambda i: (0, 0)),
        ],
        out_specs=pl.BlockSpec((_BA, NN), lambda i: (i, 0)),
        out_shape=jax.ShapeDtypeStruct((NN, NN), jnp.float32),
    )(s, s)


def kernel(x, edge_index, W1e, b1e, W2e, b2e, Wa1, ba1, Wa2, ba2, Ws1, bs1):
    src = edge_index[0].astype(jnp.int32)
    dst = edge_index[1].astype(jnp.int32)
    pad = EP - EE
    src_p = jnp.concatenate([src, jnp.zeros((pad,), jnp.int32)])
    dst_p = jnp.concatenate([dst, jnp.full((pad,), NN, jnp.int32)])
    srcs = jnp.stack([src_p * 2, src_p * 2 + 1]).reshape(2, 16, NB, B)
    dst3 = dst_p.reshape(16, NB, B)
    dstd = dst_p.reshape(32, NBD, B)
    xp = jnp.pad(x, ((0, NP - NN), (0, 0)))
    zeros = jnp.zeros((CH, HF), jnp.float32)
    ones16 = jnp.ones((B, HD), jnp.float32)
    zeros16 = jnp.zeros((RPT, HD), jnp.float32)

    def agg(zp):
        a = _agg_call(srcs, dst3, zp.reshape(2 * NP, HF), zeros)
        return a.reshape(NP, FE)

    degp = _deg_call(dstd, ones16, zeros16)
    deg = degp[:, 0, 0] + degp[:, 1, 0] + 1.0
    dinv = lax.rsqrt(jnp.maximum(deg, 1.0)).reshape(NP, 1)

    zp1 = _prep_call(xp, W1e, dinv)
    zp2 = _next_call(agg(zp1), zp1, dinv, b1e.reshape(1, FE), W2e)
    agg2 = agg(zp2)
    zp5 = _next_call(agg2, zp2, dinv, b2e.reshape(1, FE), Ws1)
    zp3 = _next_call(agg2, zp2, dinv, b2e.reshape(1, FE), Wa1)
    s = _fin_call(agg(zp5), zp5, dinv, bs1.reshape(1, FE))
    A_hat = _rec_call(s)
    zp4 = _next_call(agg(zp3), zp3, dinv, ba1.reshape(1, FE), Wa2)
    X_hat = _fin_call(agg(zp4), zp4, dinv, ba2.reshape(1, FE))
    return (A_hat, X_hat)


# speedup vs baseline: 1.0696x; 1.0199x over previous
import functools

import jax
import jax.numpy as jnp
from jax import lax
from jax.experimental import pallas as pl
from jax.experimental.pallas import tpu as pltpu
from jax.experimental.pallas import tpu_sc as plsc

NN = 10000
FE = 128
HF = 64
EE = 320000
NP = 10240
EP = 323584
B = 128
NB = 158
KR = 3
RPT = NP // 16
CH = RPT // 2

_MESH = dict(core_axis_name="c", subcore_axis_name="s")


def _agg_body(src_hbm, dst_hbm, zpf_hbm, zeros_hbm, out_hbm, src_v, dst_v,
              row_v, buf_v, acc_sh, sem):
    c = lax.axis_index("c")
    t = lax.axis_index("s")
    pltpu.sync_copy(src_hbm.at[c, t], src_v)
    pltpu.sync_copy(dst_hbm.at[t], dst_v)
    pltpu.sync_copy(zeros_hbm, buf_v)
    for k in range(2):
        pltpu.sync_copy(buf_v, acc_sh.at[pl.ds(t * RPT + k * CH, CH)])
    plsc.subcore_barrier()

    NBR = (NB // KR) * KR
    for b in range(KR):
        pltpu.async_copy(zpf_hbm.at[src_v.at[b]], row_v.at[b], sem.at[b])

    def group(g, carry):
        for b in range(KR):
            j = g * KR + b
            pltpu.make_async_copy(
                zpf_hbm.at[src_v.at[j]], row_v.at[b], sem.at[b]).wait()
            pltpu.sync_copy(row_v.at[b], acc_sh.at[dst_v.at[j]], add=True)

            @pl.when(j + KR < NBR)
            def _():
                pltpu.async_copy(
                    zpf_hbm.at[src_v.at[j + KR]], row_v.at[b], sem.at[b])
        return carry

    lax.fori_loop(0, NB // KR, group, 0)
    for j in range(NBR, NB):
        pltpu.sync_copy(zpf_hbm.at[src_v.at[j]], row_v.at[0])
        pltpu.sync_copy(row_v.at[0], acc_sh.at[dst_v.at[j]], add=True)
    plsc.subcore_barrier()
    for k in range(2):
        sl = pl.ds(t * RPT + k * CH, CH)
        pltpu.sync_copy(acc_sh.at[sl], buf_v)
        pltpu.sync_copy(buf_v, out_hbm.at[sl, c])


@functools.partial(
    pl.kernel,
    mesh=plsc.VectorSubcoreMesh(**_MESH),
    out_type=jax.ShapeDtypeStruct((NP, 2, HF), jnp.float32),
    scratch_types=[
        pltpu.VMEM((NB, B), jnp.int32),
        pltpu.VMEM((NB, B), jnp.int32),
        pltpu.VMEM((KR, B, HF), jnp.float32),
        pltpu.VMEM((CH, HF), jnp.float32),
        pltpu.VMEM_SHARED((NP, HF), jnp.float32),
        pltpu.SemaphoreType.DMA((KR,)),
    ],
    compiler_params=pltpu.CompilerParams(use_tc_tiling_on_sc=False),
)
def _agg_call(*args):
    _agg_body(*args)


NBD = EP // (32 * B)
HD = 16


def _deg_body(dst_hbm, ones_hbm, zeros_hbm, out_hbm, dst_v, ones_v, buf_v,
              acc_sh):
    c = lax.axis_index("c")
    t = lax.axis_index("s")
    wid = t * 2 + c
    pltpu.sync_copy(dst_hbm.at[wid], dst_v)
    pltpu.sync_copy(ones_hbm, ones_v)
    pltpu.sync_copy(zeros_hbm, buf_v)
    pltpu.sync_copy(buf_v, acc_sh.at[pl.ds(t * RPT, RPT)])
    plsc.subcore_barrier()

    def body(j, carry):
        pltpu.sync_copy(ones_v, acc_sh.at[dst_v.at[j]], add=True)
        return carry

    lax.fori_loop(0, NBD, body, 0)
    plsc.subcore_barrier()
    sl = pl.ds(t * RPT, RPT)
    pltpu.sync_copy(acc_sh.at[sl], buf_v)
    pltpu.sync_copy(buf_v, out_hbm.at[sl, c])


@functools.partial(
    pl.kernel,
    mesh=plsc.VectorSubcoreMesh(**_MESH),
    out_type=jax.ShapeDtypeStruct((NP, 2, HD), jnp.float32),
    scratch_types=[
        pltpu.VMEM((NBD, B), jnp.int32),
        pltpu.VMEM((B, HD), jnp.float32),
        pltpu.VMEM((RPT, HD), jnp.float32),
        pltpu.VMEM_SHARED((NP, HD), jnp.float32),
    ],
    compiler_params=pltpu.CompilerParams(use_tc_tiling_on_sc=False),
)
def _deg_call(*args):
    _deg_body(*args)


_BM = 1280


def _prep_body(xp_ref, w_ref, dinv_ref, zp_ref):
    z = jnp.dot(xp_ref[...], w_ref[...],
                preferred_element_type=jnp.float32,
                precision=lax.Precision.HIGHEST)
    zp_ref[...] = z * dinv_ref[...]


def _prep_call(xp, w, dinv):
    return pl.pallas_call(
        _prep_body,
        grid=(NP // _BM,),
        in_specs=[
            pl.BlockSpec((_BM, FE), lambda i: (i, 0)),
            pl.BlockSpec((FE, FE), lambda i: (0, 0)),
            pl.BlockSpec((_BM, 1), lambda i: (i, 0)),
        ],
        out_specs=pl.BlockSpec((_BM, FE), lambda i: (i, 0)),
        out_shape=jax.ShapeDtypeStruct((NP, FE), jnp.float32),
    )(xp, w, dinv)


def _next_body(agg_ref, zp_ref, dinv_ref, b_ref, w_ref, out_ref):
    dinv = dinv_ref[...]
    h = jnp.maximum((agg_ref[...] + zp_ref[...]) * dinv + b_ref[...], 0.0)
    z = jnp.dot(h, w_ref[...], preferred_element_type=jnp.float32,
                precision=lax.Precision.HIGHEST)
    out_ref[...] = z * dinv


def _next_call(agg, zp, dinv, b, w):
    return pl.pallas_call(
        _next_body,
        grid=(NP // _BM,),
        in_specs=[
            pl.BlockSpec((_BM, FE), lambda i: (i, 0)),
            pl.BlockSpec((_BM, FE), lambda i: (i, 0)),
            pl.BlockSpec((_BM, 1), lambda i: (i, 0)),
            pl.BlockSpec((1, FE), lambda i: (0, 0)),
            pl.BlockSpec((FE, FE), lambda i: (0, 0)),
        ],
        out_specs=pl.BlockSpec((_BM, FE), lambda i: (i, 0)),
        out_shape=jax.ShapeDtypeStruct((NP, FE), jnp.float32),
    )(agg, zp, dinv, b, w)


_BMF = 1000


def _fin_body(agg_ref, zp_ref, dinv_ref, b_ref, out_ref):
    h = (agg_ref[...] + zp_ref[...]) * dinv_ref[...] + b_ref[...]
    out_ref[...] = jnp.maximum(h, 0.0)


def _fin_call(agg, zp, dinv, b):
    return pl.pallas_call(
        _fin_body,
        grid=(NN // _BMF,),
        in_specs=[
            pl.BlockSpec((_BMF, FE), lambda i: (i, 0)),
            pl.BlockSpec((_BMF, FE), lambda i: (i, 0)),
            pl.BlockSpec((_BMF, 1), lambda i: (i, 0)),
            pl.BlockSpec((1, FE), lambda i: (0, 0)),
        ],
        out_specs=pl.BlockSpec((_BMF, FE), lambda i: (i, 0)),
        out_shape=jax.ShapeDtypeStruct((NN, FE), jnp.float32),
    )(agg, zp, dinv, b)


_BA = 400


def _rec_body(si_ref, sj_ref, out_ref):
    out_ref[...] = lax.dot_general(
        si_ref[...], sj_ref[...], (((1,), (1,)), ((), ())),
        precision=lax.Precision.DEFAULT,
        preferred_element_type=jnp.float32)


def _rec_call(s):
    return pl.pallas_call(
        _rec_body,
        grid=(NN // _BA,),
        in_specs=[
            pl.BlockSpec((_BA, FE), lambda i: (i, 0)),
            pl.BlockSpec((NN, FE), lambda i: (0, 0)),
        ],
        out_specs=pl.BlockSpec((_BA, NN), lambda i: (i, 0)),
        out_shape=jax.ShapeDtypeStruct((NN, NN), jnp.float32),
    )(s, s)


def kernel(x, edge_index, W1e, b1e, W2e, b2e, Wa1, ba1, Wa2, ba2, Ws1, bs1):
    src = edge_index[0].astype(jnp.int32)
    dst = edge_index[1].astype(jnp.int32)
    pad = EP - EE
    src_p = jnp.concatenate([src, jnp.zeros((pad,), jnp.int32)])
    dst_p = jnp.concatenate([dst, jnp.full((pad,), NN, jnp.int32)])
    srcs = jnp.stack([src_p * 2, src_p * 2 + 1]).reshape(2, 16, NB, B)
    dst3 = dst_p.reshape(16, NB, B)
    dstd = dst_p.reshape(32, NBD, B)
    xp = jnp.pad(x, ((0, NP - NN), (0, 0)))
    zeros = jnp.zeros((CH, HF), jnp.float32)
    ones16 = jnp.ones((B, HD), jnp.float32)
    zeros16 = jnp.zeros((RPT, HD), jnp.float32)

    def agg(zp):
        a = _agg_call(srcs, dst3, zp.reshape(2 * NP, HF), zeros)
        return a.reshape(NP, FE)

    degp = _deg_call(dstd, ones16, zeros16)
    deg = degp[:, 0, 0] + degp[:, 1, 0] + 1.0
    dinv = lax.rsqrt(jnp.maximum(deg, 1.0)).reshape(NP, 1)

    zp1 = _prep_call(xp, W1e, dinv)
    zp2 = _next_call(agg(zp1), zp1, dinv, b1e.reshape(1, FE), W2e)
    agg2 = agg(zp2)
    zp5 = _next_call(agg2, zp2, dinv, b2e.reshape(1, FE), Ws1)
    zp3 = _next_call(agg2, zp2, dinv, b2e.reshape(1, FE), Wa1)
    s = _fin_call(agg(zp5), zp5, dinv, bs1.reshape(1, FE))
    A_hat = _rec_call(s)
    zp4 = _next_call(agg(zp3), zp3, dinv, ba1.reshape(1, FE), Wa2)
    X_hat = _fin_call(agg(zp4), zp4, dinv, ba2.reshape(1, FE))
    return (A_hat, X_hat)
